# baseline (speedup 1.0000x reference)
import functools

import jax
import jax.numpy as jnp
from jax import lax
from jax.experimental import pallas as pl
from jax.experimental.pallas import tpu as pltpu
from jax.experimental.pallas import tpu_sc as plsc

L = 128
E = 320000
N = 10000
BE = 1280
BN = 2000
NEB = E // BE
NNB = N // BN


def _dot(a, b):
    return lax.dot_general(a, b, (((1,), (0,)), ((), ())),
                           preferred_element_type=jnp.float32)


def _red8(v):
    acc = v[:, 0:8]
    for i in range(1, 16):
        acc = acc + v[:, 8 * i:8 * i + 8]
    f = acc[:, 0:4] + acc[:, 4:8]
    f = f[:, 0:2] + f[:, 2:4]
    return f[:, 0:1] + f[:, 1:2]


def _mlp3(xcat, Wp, vp, d_in, ln=True):
    W1 = Wp[0:d_in]
    W2 = Wp[d_in:d_in + L]
    W3 = Wp[d_in + L:d_in + 2 * L]
    h = jnp.maximum(_dot(xcat, W1) + vp[0:1], 0.0)
    h = jnp.maximum(_dot(h, W2) + vp[1:2], 0.0)
    y = _dot(h, W3) + vp[2:3]
    if ln:
        mu = _red8(y) * (1.0 / 128.0)
        c = y - mu
        var = _red8(c * c) * (1.0 / 128.0)
        y = c / jnp.sqrt(var + 1e-5) * vp[3:4] + vp[4:5]
    return y


def _enc_body(x_ref, Wp_ref, vp_ref, o_ref, *, d_in, ln):
    o_ref[...] = _mlp3(x_ref[...], Wp_ref[...], vp_ref[...], d_in, ln=ln)


def _edge_body2(hs_ref, hd_ref, he_ref, Wp_ref, vp_ref, e_ref, hen_ref):
    xcat = jnp.concatenate([hs_ref[...], hd_ref[...], he_ref[...]], axis=-1)
    e = _mlp3(xcat, Wp_ref[...], vp_ref[...], 3 * L)
    e_ref[...] = e
    hen_ref[...] = e + he_ref[...]


def _edge_body1(hs_ref, hd_ref, he_ref, Wp_ref, vp_ref, e_ref):
    xcat = jnp.concatenate([hs_ref[...], hd_ref[...], he_ref[...]], axis=-1)
    e_ref[...] = _mlp3(xcat, Wp_ref[...], vp_ref[...], 3 * L)


def _node_body(hx_ref, p0_ref, p1_ref, Wp_ref, vp_ref, o_ref):
    agg = p0_ref[0] + p1_ref[0]
    xcat = jnp.concatenate([hx_ref[...], agg], axis=-1)
    o_ref[...] = _mlp3(xcat, Wp_ref[...], vp_ref[...], 2 * L) + hx_ref[...]


def _row_spec(b, cols):
    return pl.BlockSpec((b, cols), lambda i: (i, 0))


def _w_spec(rows):
    return pl.BlockSpec((rows, L), lambda i: (0, 0))


def _enc_call(x, Wp, vp, d_in, ln=True):
    n = x.shape[0]
    b = BE if n == E else BN
    return pl.pallas_call(
        functools.partial(_enc_body, d_in=d_in, ln=ln),
        grid=(n // b,),
        in_specs=[_row_spec(b, d_in), _w_spec(d_in + 2 * L),
                  pl.BlockSpec((8, L), lambda i: (0, 0))],
        out_specs=_row_spec(b, L),
        out_shape=jax.ShapeDtypeStruct((n, L), jnp.float32),
    )(x, Wp, vp)


def _edge_call(g, he, Wp, vp, want_he):
    body = _edge_body2 if want_he else _edge_body1
    out_shape = [jax.ShapeDtypeStruct((E, L), jnp.float32)]
    out_specs = [_row_spec(BE, L)]
    if want_he:
        out_shape.append(jax.ShapeDtypeStruct((E, L), jnp.float32))
        out_specs.append(_row_spec(BE, L))
    outs = pl.pallas_call(
        body,
        grid=(NEB,),
        in_specs=[
            pl.BlockSpec((BE, L), lambda i: (i, 0)),
            pl.BlockSpec((BE, L), lambda i: (i + NEB, 0)),
            _row_spec(BE, L),
            _w_spec(5 * L),
            pl.BlockSpec((8, L), lambda i: (0, 0)),
        ],
        out_specs=out_specs,
        out_shape=out_shape,
    )(g, g, he, Wp, vp)
    return outs if want_he else (outs[0], None)


def _node_call(hx, p01, Wp, vp):
    return pl.pallas_call(
        _node_body,
        grid=(NNB,),
        in_specs=[
            _row_spec(BN, L),
            pl.BlockSpec((1, BN, L), lambda i: (0, i, 0)),
            pl.BlockSpec((1, BN, L), lambda i: (1, i, 0)),
            _w_spec(4 * L),
            pl.BlockSpec((8, L), lambda i: (0, 0)),
        ],
        out_specs=_row_spec(BN, L),
        out_shape=jax.ShapeDtypeStruct((N, L), jnp.float32),
    )(hx, p01, p01, Wp, vp)


NC, NS = 2, 16
NW = NC * NS
_SC_MESH = plsc.VectorSubcoreMesh(core_axis_name="c", subcore_axis_name="s")

GCH = 128
NGCH = 2 * E // GCH
GITER = (NGCH + NW - 1) // NW

ECORE = E // NC
SCH = 128
NSCH = ECORE // SCH
SITER = (NSCH + NS - 1) // NS
ZCH = 80
NZCH = N // ZCH
ZITER = (NZCH + NS - 1) // NS


@functools.partial(
    pl.kernel,
    out_type=jax.ShapeDtypeStruct((2 * E, L), jnp.float32),
    mesh=_SC_MESH,
    scratch_types=[pltpu.VMEM((GCH,), jnp.int32),
                   pltpu.VMEM((GCH, L), jnp.float32),
                   pltpu.SemaphoreType.DMA],
)
def _gather_sc(hx_hbm, idx_hbm, out_hbm, idx_v, rows_v, sem):
    wid = lax.axis_index("s") * NC + lax.axis_index("c")

    def body(j, carry):
        ch = wid + NW * j

        @pl.when(ch < NGCH)
        def _():
            base = ch * GCH
            pltpu.sync_copy(idx_hbm.at[pl.ds(base, GCH)], idx_v)
            pltpu.async_copy(hx_hbm.at[idx_v], rows_v, sem).wait()
            pltpu.sync_copy(rows_v, out_hbm.at[pl.ds(base, GCH)])
        return carry

    lax.fori_loop(0, GITER, body, 0)


@functools.partial(
    pl.kernel,
    out_type=jax.ShapeDtypeStruct((NC, N, L), jnp.float32),
    mesh=_SC_MESH,
    scratch_types=[pltpu.VMEM((1, SCH), jnp.int32),
                   pltpu.VMEM((SCH, L), jnp.float32),
                   pltpu.VMEM((ZCH, L), jnp.float32),
                   pltpu.VMEM_SHARED((N, L), jnp.float32)],
)
def _scatter_sc(e_hbm, idx_hbm, out_hbm, idx_v, rows_v, stage_v, acc_sh):
    c = lax.axis_index("c")
    s = lax.axis_index("s")

    def zb(i, carry):
        for jj in range(L // 16):
            stage_v[i, pl.ds(jj * 16, 16)] = jnp.zeros((16,), jnp.float32)
        return carry

    lax.fori_loop(0, ZCH, zb, 0)
    for k in range(ZITER):
        ch0 = s + NS * k

        @pl.when(ch0 < NZCH)
        def _():
            pltpu.sync_copy(stage_v, acc_sh.at[pl.ds(ch0 * ZCH, ZCH)])
    plsc.subcore_barrier()

    @pl.when((c == 0) & (s == 0))
    def _():
        def body(ch, carry):
            base = ch * SCH
            pltpu.sync_copy(idx_hbm.at[pl.ds(E + base, SCH)], idx_v.at[0])
            pltpu.sync_copy(e_hbm.at[pl.ds(base, SCH)], rows_v)
            pltpu.sync_copy(rows_v, acc_sh.at[idx_v.at[0]], add=True)
            return carry

        lax.fori_loop(0, E // SCH, body, 0)
    plsc.subcore_barrier()

    for k in range(ZITER):
        ch1 = s + NS * k

        @pl.when(ch1 < NZCH)
        def _():
            pltpu.sync_copy(acc_sh.at[pl.ds(ch1 * ZCH, ZCH)], stage_v)
            pltpu.sync_copy(stage_v, out_hbm.at[c, pl.ds(ch1 * ZCH, ZCH)])


def _gather(hx, idx):
    return _gather_sc(hx, idx)


def _scatter(e_new, idx):
    return _scatter_sc(e_new, idx)


def _pack(p, ln=True):
    ws = [q["W"] for q in p["mlp"]] if ln else [q["W"] for q in p]
    bs = [q["b"] for q in p["mlp"]] if ln else [q["b"] for q in p]
    if ws[-1].shape[1] != L:
        ws = ws[:-1] + [jnp.pad(ws[-1], ((0, 0), (0, L - ws[-1].shape[1])))]
        bs = bs[:-1] + [jnp.pad(bs[-1], (0, L - bs[-1].shape[0]))]
    Wp = jnp.concatenate(ws, axis=0)
    rows = bs + ([p["ln_g"], p["ln_b"]] if ln else [])
    vp = jnp.stack(rows)
    vp = jnp.pad(vp, ((0, 8 - vp.shape[0]), (0, 0)))
    return Wp, vp


def kernel(x, edge_attr, edge_index, params):
    Wn, vn = _pack(params["enc_node"])
    We, ve = _pack(params["enc_edge"])
    Wd, vd = _pack(params["dec"], ln=False)

    hx = _enc_call(x, Wn, vn, L)
    he = _enc_call(edge_attr, We, ve, 16)
    idx = edge_index.reshape(-1)

    for s, step in enumerate(params["proc"]):
        Wep, vep = _pack(step["edge"])
        Wnp, vnp = _pack(step["node"])
        g = _gather(hx, idx)
        e_new, he_new = _edge_call(g, he, Wep, vep, want_he=(s < 4))
        p01 = _scatter(e_new, idx)
        hx = _node_call(hx, p01, Wnp, vnp)
        he = he_new

    out = _enc_call(hx, Wd, vd, L, ln=False)
    return out[:, :3]


# speedup vs baseline: 1.6721x; 1.6721x over previous
import functools

import jax
import jax.numpy as jnp
from jax import lax
from jax.experimental import pallas as pl
from jax.experimental.pallas import tpu as pltpu
from jax.experimental.pallas import tpu_sc as plsc

L = 128
E = 320000
N = 10000
BE = 1280
BN = 2000
NEB = E // BE
NNB = N // BN


def _dot(a, b):
    return lax.dot_general(a, b, (((1,), (0,)), ((), ())),
                           preferred_element_type=jnp.float32)


def _red8(v):
    acc = v[:, 0:8]
    for i in range(1, 16):
        acc = acc + v[:, 8 * i:8 * i + 8]
    f = acc[:, 0:4] + acc[:, 4:8]
    f = f[:, 0:2] + f[:, 2:4]
    return f[:, 0:1] + f[:, 1:2]


def _mlp3(xcat, Wp, vp, d_in, ln=True):
    W1 = Wp[0:d_in]
    W2 = Wp[d_in:d_in + L]
    W3 = Wp[d_in + L:d_in + 2 * L]
    h = jnp.maximum(_dot(xcat, W1) + vp[0:1], 0.0)
    h = jnp.maximum(_dot(h, W2) + vp[1:2], 0.0)
    y = _dot(h, W3) + vp[2:3]
    if ln:
        mu = _red8(y) * (1.0 / 128.0)
        c = y - mu
        var = _red8(c * c) * (1.0 / 128.0)
        y = c / jnp.sqrt(var + 1e-5) * vp[3:4] + vp[4:5]
    return y


def _enc_body(x_ref, Wp_ref, vp_ref, o_ref, *, d_in, ln):
    o_ref[...] = _mlp3(x_ref[...], Wp_ref[...], vp_ref[...], d_in, ln=ln)


def _edge_body2(hs_ref, hd_ref, he_ref, Wp_ref, vp_ref, e_ref, hen_ref):
    xcat = jnp.concatenate([hs_ref[...], hd_ref[...], he_ref[...]], axis=-1)
    e = _mlp3(xcat, Wp_ref[...], vp_ref[...], 3 * L)
    e_ref[...] = e
    hen_ref[...] = e + he_ref[...]


def _edge_body1(hs_ref, hd_ref, he_ref, Wp_ref, vp_ref, e_ref):
    xcat = jnp.concatenate([hs_ref[...], hd_ref[...], he_ref[...]], axis=-1)
    e_ref[...] = _mlp3(xcat, Wp_ref[...], vp_ref[...], 3 * L)


def _node_body(hx_ref, p0_ref, Wp_ref, vp_ref, o_ref):
    xcat = jnp.concatenate([hx_ref[...], p0_ref[...]], axis=-1)
    o_ref[...] = _mlp3(xcat, Wp_ref[...], vp_ref[...], 2 * L) + hx_ref[...]


def _row_spec(b, cols):
    return pl.BlockSpec((b, cols), lambda i: (i, 0))


def _w_spec(rows):
    return pl.BlockSpec((rows, L), lambda i: (0, 0))


def _enc_call(x, Wp, vp, d_in, ln=True):
    n = x.shape[0]
    b = BE if n == E else BN
    return pl.pallas_call(
        functools.partial(_enc_body, d_in=d_in, ln=ln),
        grid=(n // b,),
        in_specs=[_row_spec(b, d_in), _w_spec(d_in + 2 * L),
                  pl.BlockSpec((8, L), lambda i: (0, 0))],
        out_specs=_row_spec(b, L),
        out_shape=jax.ShapeDtypeStruct((n, L), jnp.float32),
    )(x, Wp, vp)


def _edge_call(g, he, Wp, vp, want_he):
    body = _edge_body2 if want_he else _edge_body1
    out_shape = [jax.ShapeDtypeStruct((E, L), jnp.float32)]
    out_specs = [_row_spec(BE, L)]
    if want_he:
        out_shape.append(jax.ShapeDtypeStruct((E, L), jnp.float32))
        out_specs.append(_row_spec(BE, L))
    outs = pl.pallas_call(
        body,
        grid=(NEB,),
        in_specs=[
            pl.BlockSpec((BE, L), lambda i: (i, 0)),
            pl.BlockSpec((BE, L), lambda i: (i + NEB, 0)),
            _row_spec(BE, L),
            _w_spec(5 * L),
            pl.BlockSpec((8, L), lambda i: (0, 0)),
        ],
        out_specs=out_specs,
        out_shape=out_shape,
    )(g, g, he, Wp, vp)
    return outs if want_he else (outs[0], None)


def _node_call(hx, p0, Wp, vp):
    return pl.pallas_call(
        _node_body,
        grid=(NNB,),
        in_specs=[
            _row_spec(BN, L),
            _row_spec(BN, L),
            _w_spec(4 * L),
            pl.BlockSpec((8, L), lambda i: (0, 0)),
        ],
        out_specs=_row_spec(BN, L),
        out_shape=jax.ShapeDtypeStruct((N, L), jnp.float32),
    )(hx, p0, Wp, vp)


NC, NS = 2, 16
NW = NC * NS
_SC_MESH = plsc.VectorSubcoreMesh(core_axis_name="c", subcore_axis_name="s")

GCH = 128
NGCH = 2 * E // GCH
GITER = (NGCH + NW - 1) // NW

ECORE = E // NC
SCH = 128
NSCH = ECORE // SCH
SITER = (NSCH + NS - 1) // NS
ZCH = 80
NZCH = N // ZCH
ZITER = (NZCH + NS - 1) // NS


@functools.partial(
    pl.kernel,
    out_type=jax.ShapeDtypeStruct((2 * E, L), jnp.float32),
    mesh=_SC_MESH,
    scratch_types=[pltpu.VMEM((GCH,), jnp.int32),
                   pltpu.VMEM((GCH, L), jnp.float32),
                   pltpu.SemaphoreType.DMA],
)
def _gather_sc(hx_hbm, idx_hbm, out_hbm, idx_v, rows_v, sem):
    wid = lax.axis_index("s") * NC + lax.axis_index("c")

    def body(j, carry):
        ch = wid + NW * j

        @pl.when(ch < NGCH)
        def _():
            base = ch * GCH
            pltpu.sync_copy(idx_hbm.at[pl.ds(base, GCH)], idx_v)
            pltpu.async_copy(hx_hbm.at[idx_v], rows_v, sem).wait()
            pltpu.sync_copy(rows_v, out_hbm.at[pl.ds(base, GCH)])
        return carry

    lax.fori_loop(0, GITER, body, 0)


NCH2 = E // SCH


@functools.partial(
    pl.kernel,
    out_type=jax.ShapeDtypeStruct((N, L), jnp.float32),
    mesh=_SC_MESH,
    scratch_types=[pltpu.VMEM((2, SCH), jnp.int32),
                   pltpu.VMEM((2, SCH, L), jnp.float32),
                   pltpu.VMEM((ZCH, L), jnp.float32),
                   pltpu.VMEM_SHARED((N, L), jnp.float32),
                   pltpu.SemaphoreType.DMA,
                   pltpu.SemaphoreType.DMA],
)
def _scatter_sc(e_hbm, idx_hbm, out_hbm, idx_v, rows_v, stage_v, acc_sh,
                sem0, sem1):
    c = lax.axis_index("c")
    s = lax.axis_index("s")
    sems = (sem0, sem1)

    @pl.when(c == 0)
    def _():
        def zb(i, carry):
            for jj in range(L // 16):
                stage_v[i, pl.ds(jj * 16, 16)] = jnp.zeros((16,), jnp.float32)
            return carry

        lax.fori_loop(0, ZCH, zb, 0)
        for k in range(ZITER):
            ch0 = s + NS * k

            @pl.when(ch0 < NZCH)
            def _():
                pltpu.sync_copy(stage_v, acc_sh.at[pl.ds(ch0 * ZCH, ZCH)])
    plsc.subcore_barrier()

    @pl.when((c == 0) & (s == 0))
    def _():
        def start(ch, b):
            base = ch * SCH
            pltpu.async_copy(idx_hbm.at[pl.ds(E + base, SCH)], idx_v.at[b],
                             sems[b])
            pltpu.async_copy(e_hbm.at[pl.ds(base, SCH)], rows_v.at[b], sems[b])

        def wait(b):
            pltpu.make_async_copy(idx_hbm.at[pl.ds(0, SCH)], idx_v.at[b],
                                  sems[b]).wait()
            pltpu.make_async_copy(e_hbm.at[pl.ds(0, SCH)], rows_v.at[b],
                                  sems[b]).wait()

        start(0, 0)

        def outer(g, carry):
            for b in range(2):
                ch = 2 * g + b

                @pl.when(ch + 1 < NCH2)
                def _():
                    start(ch + 1, 1 - b)
                wait(b)
                pltpu.sync_copy(rows_v.at[b], acc_sh.at[idx_v.at[b]], add=True)
            return carry

        lax.fori_loop(0, NCH2 // 2, outer, 0)
    plsc.subcore_barrier()

    @pl.when(c == 0)
    def _():
        for k in range(ZITER):
            ch1 = s + NS * k

            @pl.when(ch1 < NZCH)
            def _():
                pltpu.sync_copy(acc_sh.at[pl.ds(ch1 * ZCH, ZCH)], stage_v)
                pltpu.sync_copy(stage_v, out_hbm.at[pl.ds(ch1 * ZCH, ZCH)])


def _gather(hx, idx):
    return _gather_sc(hx, idx)


def _scatter(e_new, idx):
    return _scatter_sc(e_new, idx)


def _pack(p, ln=True):
    ws = [q["W"] for q in p["mlp"]] if ln else [q["W"] for q in p]
    bs = [q["b"] for q in p["mlp"]] if ln else [q["b"] for q in p]
    if ws[-1].shape[1] != L:
        ws = ws[:-1] + [jnp.pad(ws[-1], ((0, 0), (0, L - ws[-1].shape[1])))]
        bs = bs[:-1] + [jnp.pad(bs[-1], (0, L - bs[-1].shape[0]))]
    Wp = jnp.concatenate(ws, axis=0)
    rows = bs + ([p["ln_g"], p["ln_b"]] if ln else [])
    vp = jnp.stack(rows)
    vp = jnp.pad(vp, ((0, 8 - vp.shape[0]), (0, 0)))
    return Wp, vp


def kernel(x, edge_attr, edge_index, params):
    Wn, vn = _pack(params["enc_node"])
    We, ve = _pack(params["enc_edge"])
    Wd, vd = _pack(params["dec"], ln=False)

    hx = _enc_call(x, Wn, vn, L)
    he = _enc_call(edge_attr, We, ve, 16)
    idx = edge_index.reshape(-1)

    for s, step in enumerate(params["proc"]):
        Wep, vep = _pack(step["edge"])
        Wnp, vnp = _pack(step["node"])
        g = _gather(hx, idx)
        e_new, he_new = _edge_call(g, he, Wep, vep, want_he=(s < 4))
        p0 = _scatter(e_new, idx)
        hx = _node_call(hx, p0, Wnp, vnp)
        he = he_new

    out = _enc_call(hx, Wd, vd, L, ln=False)
    return out[:, :3]


# speedup vs baseline: 2.2572x; 1.3500x over previous
import functools

import jax
import jax.numpy as jnp
from jax import lax
from jax.experimental import pallas as pl
from jax.experimental.pallas import tpu as pltpu
from jax.experimental.pallas import tpu_sc as plsc

L = 128
E = 320000
N = 10000
BE = 1280
BN = 2000
NEB = E // BE
NNB = N // BN


def _dot(a, b):
    return lax.dot_general(a, b, (((1,), (0,)), ((), ())),
                           preferred_element_type=jnp.float32)


def _red8(v):
    acc = v[:, 0:8]
    for i in range(1, 16):
        acc = acc + v[:, 8 * i:8 * i + 8]
    f = acc[:, 0:4] + acc[:, 4:8]
    f = f[:, 0:2] + f[:, 2:4]
    return f[:, 0:1] + f[:, 1:2]


def _mlp3(xcat, Wp, vp, d_in, ln=True):
    W1 = Wp[0:d_in]
    W2 = Wp[d_in:d_in + L]
    W3 = Wp[d_in + L:d_in + 2 * L]
    h = jnp.maximum(_dot(xcat, W1) + vp[0:1], 0.0)
    h = jnp.maximum(_dot(h, W2) + vp[1:2], 0.0)
    y = _dot(h, W3) + vp[2:3]
    if ln:
        mu = _red8(y) * (1.0 / 128.0)
        c = y - mu
        var = _red8(c * c) * (1.0 / 128.0)
        y = c / jnp.sqrt(var + 1e-5) * vp[3:4] + vp[4:5]
    return y


def _enc_body(x_ref, Wp_ref, vp_ref, o_ref, *, d_in, ln):
    o_ref[...] = _mlp3(x_ref[...], Wp_ref[...], vp_ref[...], d_in, ln=ln)


def _edge_body2(hs_ref, hd_ref, he_ref, Wp_ref, vp_ref, e_ref, hen_ref):
    xcat = jnp.concatenate([hs_ref[...], hd_ref[...], he_ref[...]], axis=-1)
    e = _mlp3(xcat, Wp_ref[...], vp_ref[...], 3 * L)
    e_ref[...] = e
    hen_ref[...] = e + he_ref[...]


def _edge_body1(hs_ref, hd_ref, he_ref, Wp_ref, vp_ref, e_ref):
    xcat = jnp.concatenate([hs_ref[...], hd_ref[...], he_ref[...]], axis=-1)
    e_ref[...] = _mlp3(xcat, Wp_ref[...], vp_ref[...], 3 * L)


def _node_body(hx_ref, p0_ref, Wp_ref, vp_ref, o_ref):
    xcat = jnp.concatenate([hx_ref[...], p0_ref[...]], axis=-1)
    o_ref[...] = _mlp3(xcat, Wp_ref[...], vp_ref[...], 2 * L) + hx_ref[...]


def _row_spec(b, cols):
    return pl.BlockSpec((b, cols), lambda i: (i, 0))


def _w_spec(rows):
    return pl.BlockSpec((rows, L), lambda i: (0, 0))


def _enc_call(x, Wp, vp, d_in, ln=True):
    n = x.shape[0]
    b = BE if n == E else BN
    return pl.pallas_call(
        functools.partial(_enc_body, d_in=d_in, ln=ln),
        grid=(n // b,),
        in_specs=[_row_spec(b, d_in), _w_spec(d_in + 2 * L),
                  pl.BlockSpec((8, L), lambda i: (0, 0))],
        out_specs=_row_spec(b, L),
        out_shape=jax.ShapeDtypeStruct((n, L), jnp.float32),
    )(x, Wp, vp)


def _edge_call(g, he, Wp, vp, want_he):
    body = _edge_body2 if want_he else _edge_body1
    out_shape = [jax.ShapeDtypeStruct((E, L), jnp.float32)]
    out_specs = [_row_spec(BE, L)]
    if want_he:
        out_shape.append(jax.ShapeDtypeStruct((E, L), jnp.float32))
        out_specs.append(_row_spec(BE, L))
    outs = pl.pallas_call(
        body,
        grid=(NEB,),
        in_specs=[
            pl.BlockSpec((BE, L), lambda i: (i, 0)),
            pl.BlockSpec((BE, L), lambda i: (i + NEB, 0)),
            _row_spec(BE, L),
            _w_spec(5 * L),
            pl.BlockSpec((8, L), lambda i: (0, 0)),
        ],
        out_specs=out_specs,
        out_shape=out_shape,
    )(g, g, he, Wp, vp)
    return outs if want_he else (outs[0], None)


def _node_call(hx, p0, Wp, vp):
    return pl.pallas_call(
        _node_body,
        grid=(NNB,),
        in_specs=[
            _row_spec(BN, L),
            _row_spec(BN, L),
            _w_spec(4 * L),
            pl.BlockSpec((8, L), lambda i: (0, 0)),
        ],
        out_specs=_row_spec(BN, L),
        out_shape=jax.ShapeDtypeStruct((N, L), jnp.float32),
    )(hx, p0, Wp, vp)


NC, NS = 2, 16
NW = NC * NS
_SC_MESH = plsc.VectorSubcoreMesh(core_axis_name="c", subcore_axis_name="s")

GCH = 128
NGCH = 2 * E // GCH
GITER = (NGCH + NW - 1) // NW

ECORE = E // NC
SCH = 128
NSCH = ECORE // SCH
SITER = (NSCH + NS - 1) // NS
ZCH = 80
NZCH = N // ZCH
ZITER = (NZCH + NS - 1) // NS


@functools.partial(
    pl.kernel,
    out_type=jax.ShapeDtypeStruct((2 * E, L), jnp.float32),
    mesh=_SC_MESH,
    scratch_types=[pltpu.VMEM((GCH,), jnp.int32),
                   pltpu.VMEM((GCH, L), jnp.float32),
                   pltpu.SemaphoreType.DMA],
)
def _gather_sc(hx_hbm, idx_hbm, out_hbm, idx_v, rows_v, sem):
    wid = lax.axis_index("s") * NC + lax.axis_index("c")

    def body(j, carry):
        ch = wid + NW * j

        @pl.when(ch < NGCH)
        def _():
            base = ch * GCH
            pltpu.sync_copy(idx_hbm.at[pl.ds(base, GCH)], idx_v)
            pltpu.async_copy(hx_hbm.at[idx_v], rows_v, sem).wait()
            pltpu.sync_copy(rows_v, out_hbm.at[pl.ds(base, GCH)])
        return carry

    lax.fori_loop(0, GITER, body, 0)


NTR = 64
NA = N + ZCH
NZI = NA // ZCH
ZII = (NZI + NS - 1) // NS
P = E + NS * SCH


@functools.partial(
    pl.kernel,
    out_type=jax.ShapeDtypeStruct((N, L), jnp.float32),
    mesh=_SC_MESH,
    scratch_types=[pltpu.VMEM((1, SCH), jnp.int32),
                   pltpu.VMEM((SCH,), jnp.int32),
                   pltpu.VMEM((SCH, L), jnp.float32),
                   pltpu.VMEM((ZCH, L), jnp.float32),
                   pltpu.VMEM_SHARED((NA, L), jnp.float32),
                   pltpu.VMEM((16 * NS + 16,), jnp.int32),
                   pltpu.SemaphoreType.DMA],
)
def _scatter_sc(e_hbm, dstp_hbm, permp_hbm, meta_hbm, out_hbm,
                idx_v, prm_v, rows_v, stage_v, acc_sh, meta_v, sem):
    c = lax.axis_index("c")
    s = lax.axis_index("s")

    @pl.when(c == 0)
    def _():
        def zb(i, carry):
            for jj in range(L // 16):
                stage_v[i, pl.ds(jj * 16, 16)] = jnp.zeros((16,), jnp.float32)
            return carry

        lax.fori_loop(0, ZCH, zb, 0)
        for k in range(ZII):
            ch0 = s + NS * k

            @pl.when(ch0 < NZI)
            def _():
                pltpu.sync_copy(stage_v, acc_sh.at[pl.ds(ch0 * ZCH, ZCH)])
    plsc.subcore_barrier()

    @pl.when(c == 0)
    def _():
        pltpu.sync_copy(meta_hbm, meta_v)
        base = meta_v[pl.ds(8 * s, 16)][0]
        nch = meta_v[pl.ds(8 * (NS + s), 16)][0]

        def body(j, carry):
            o = pl.multiple_of(base + j * SCH, SCH)
            pltpu.sync_copy(dstp_hbm.at[pl.ds(o, SCH)], idx_v.at[0])
            pltpu.sync_copy(permp_hbm.at[pl.ds(o, SCH)], prm_v)
            pltpu.async_copy(e_hbm.at[prm_v], rows_v, sem).wait()
            pltpu.sync_copy(rows_v, acc_sh.at[idx_v.at[0]], add=True)
            return carry

        lax.fori_loop(0, nch, body, 0)
    plsc.subcore_barrier()

    @pl.when(c == 0)
    def _():
        for k in range(ZITER):
            ch1 = s + NS * k

            @pl.when(ch1 < NZCH)
            def _():
                pltpu.sync_copy(acc_sh.at[pl.ds(ch1 * ZCH, ZCH)], stage_v)
                pltpu.sync_copy(stage_v, out_hbm.at[pl.ds(ch1 * ZCH, ZCH)])


def _gather(hx, idx):
    return _gather_sc(hx, idx)


def _build_partition(dst):
    owner = jnp.remainder(dst, NS)
    ordk = jnp.argsort(owner, stable=True).astype(jnp.int32)
    cnt = jax.ops.segment_sum(jnp.ones((E,), jnp.int32), owner,
                              num_segments=NS)
    pad_cnt = ((cnt + SCH - 1) // SCH) * SCH
    zero1 = jnp.zeros((1,), jnp.int32)
    pad_off = jnp.concatenate([zero1, jnp.cumsum(pad_cnt)]).astype(jnp.int32)
    off = jnp.concatenate([zero1, jnp.cumsum(cnt)]).astype(jnp.int32)
    own_s = owner[ordk]
    pos = pad_off[own_s] + (jnp.arange(E, dtype=jnp.int32) - off[own_s])
    trash = (N + (jnp.arange(P, dtype=jnp.int32) % NTR)).astype(jnp.int32)
    dstp = trash.at[pos].set(dst[ordk])
    permp = jnp.zeros((P,), jnp.int32).at[pos].set(ordk)
    vals = jnp.concatenate([pad_off[:NS], pad_cnt // SCH]).astype(jnp.int32)
    meta = jnp.zeros((16 * NS + 16,), jnp.int32)
    meta = meta.at[8 * jnp.arange(2 * NS)].set(vals)
    return dstp, permp, meta


def _pack(p, ln=True):
    ws = [q["W"] for q in p["mlp"]] if ln else [q["W"] for q in p]
    bs = [q["b"] for q in p["mlp"]] if ln else [q["b"] for q in p]
    if ws[-1].shape[1] != L:
        ws = ws[:-1] + [jnp.pad(ws[-1], ((0, 0), (0, L - ws[-1].shape[1])))]
        bs = bs[:-1] + [jnp.pad(bs[-1], (0, L - bs[-1].shape[0]))]
    Wp = jnp.concatenate(ws, axis=0)
    rows = bs + ([p["ln_g"], p["ln_b"]] if ln else [])
    vp = jnp.stack(rows)
    vp = jnp.pad(vp, ((0, 8 - vp.shape[0]), (0, 0)))
    return Wp, vp


def kernel(x, edge_attr, edge_index, params):
    Wn, vn = _pack(params["enc_node"])
    We, ve = _pack(params["enc_edge"])
    Wd, vd = _pack(params["dec"], ln=False)

    hx = _enc_call(x, Wn, vn, L)
    he = _enc_call(edge_attr, We, ve, 16)
    idx = edge_index.reshape(-1)
    part = _build_partition(idx[E:])

    for s, step in enumerate(params["proc"]):
        Wep, vep = _pack(step["edge"])
        Wnp, vnp = _pack(step["node"])
        g = _gather(hx, idx)
        e_new, he_new = _edge_call(g, he, Wep, vep, want_he=(s < 4))
        p0 = _scatter_sc(e_new, *part)
        hx = _node_call(hx, p0, Wnp, vnp)
        he = he_new

    out = _enc_call(hx, Wd, vd, L, ln=False)
    return out[:, :3]
